# baseline (speedup 1.0000x reference)
import functools

import jax
import jax.numpy as jnp
from jax import lax
from jax.experimental import pallas as pl
from jax.experimental.pallas import tpu as pltpu
from jax.experimental.pallas import tpu_sc as plsc

N_BOXES = 5000
PAD_N = 5120
N_CHUNKS = PAD_N // 16
N_GROUPS = N_CHUNKS // 16
MAX_OUT = 300
KPAD = 320
OB_PAD = 1216
NUM_IMAGES = 8
NEG = -1e30
IOU_THR = 0.7


def _nms_body(probs_hbm, deltas_hbm, anchors_hbm, ob_hbm, nv_hbm,
              pbuf, dbuf, anch, y1v, x1v, y2v, x2v, arv,
              skeys, spay, cmax, ptr,
              ky1, kx1, ky2, kx2, ob, nvv, sem):
    wid = lax.axis_index("s") * 2 + lax.axis_index("c")

    @pl.when(wid < NUM_IMAGES)
    def _():
        b = wid
        pltpu.sync_copy(anchors_hbm, anch)
        pltpu.sync_copy(probs_hbm.at[b], pbuf)
        pltpu.sync_copy(deltas_hbm.at[b], dbuf)

        iota = lax.iota(jnp.int32, 16)
        zi = jnp.zeros((16,), jnp.int32)
        lane0 = iota == 0
        negvec = jnp.full((16,), NEG, jnp.float32)
        zf = jnp.zeros((16,), jnp.float32)

        def bmax(v):
            return jnp.full((16,), jnp.max(v))

        xb = plsc.load_gather(anch, [zi + 4])
        yb = plsc.load_gather(anch, [zi + 5])
        wb = plsc.load_gather(anch, [zi + 6])
        hb = plsc.load_gather(anch, [zi + 7])

        for t in range(KPAD // 16):
            ky1[pl.ds(t * 16, 16)] = zf
            kx1[pl.ds(t * 16, 16)] = zf
            ky2[pl.ds(t * 16, 16)] = zf
            kx2[pl.ds(t * 16, 16)] = zf

        def decode_body(c, _):
            i = c * 16 + iota
            im = jnp.minimum(i, N_BOXES - 1)
            sc_ = plsc.load_gather(pbuf, [im, zi + 1])
            sc_ = jnp.where(i < N_BOXES, sc_, negvec)
            xd = plsc.load_gather(dbuf, [im, zi])
            yd = plsc.load_gather(dbuf, [im, zi + 1])
            wd = plsc.load_gather(dbuf, [im, zi + 2])
            hd = plsc.load_gather(dbuf, [im, zi + 3])
            y1 = jnp.minimum(jnp.maximum(xd * wb + xb, 0.0), 1.0)
            x1 = jnp.minimum(jnp.maximum(yd * hb + yb, 0.0), 1.0)
            y2 = jnp.minimum(jnp.maximum(jnp.exp(wd) * wb, 0.0), 1.0)
            x2 = jnp.minimum(jnp.maximum(jnp.exp(hd) * hb, 0.0), 1.0)
            plsc.store_scatter(y1v, [i], y1)
            plsc.store_scatter(x1v, [i], x1)
            plsc.store_scatter(y2v, [i], y2)
            plsc.store_scatter(x2v, [i], x2)
            plsc.store_scatter(arv, [i], (y2 - y1) * (x2 - x1))
            sk, sp_ = plsc.sort_key_val(sc_, i, descending=True)
            plsc.store_scatter(skeys, [i], sk)
            plsc.store_scatter(spay, [i], sp_)
            plsc.store_scatter(cmax, [jnp.full((16,), c)], sk, mask=lane0)
            return 0

        lax.fori_loop(0, N_CHUNKS, decode_body, 0)

        for t in range(N_GROUPS, 32):
            cmax[pl.ds(t * 16, 16)] = negvec
        for t in range(N_CHUNKS // 16):
            ptr[pl.ds(t * 16, 16)] = zi

        l2 = []
        for t in range(2):
            m = plsc.load_gather(cmax, [t * 256 + 16 * iota])
            for j in range(1, 16):
                m = jnp.maximum(m, plsc.load_gather(cmax, [t * 256 + 16 * iota + j]))
            l2.append(m)

        c4 = jnp.bitwise_and(iota, 3)
        vy10 = bmax(jnp.where(lane0, y1v[pl.ds(0, 16)], NEG))
        vx10 = bmax(jnp.where(lane0, x1v[pl.ds(0, 16)], NEG))
        vy20 = bmax(jnp.where(lane0, y2v[pl.ds(0, 16)], NEG))
        vx20 = bmax(jnp.where(lane0, x2v[pl.ds(0, 16)], NEG))
        pat = jnp.where(c4 == 0, vy10,
                        jnp.where(c4 == 1, vx10,
                                  jnp.where(c4 == 2, vy20, vx20)))
        for t in range(MAX_OUT * 4 // 16):
            ob[pl.ds(t * 16, 16)] = pat

        def loop_cond(state):
            kept, alive, _, _ = state
            return (kept < MAX_OUT) & (alive == 1)

        def loop_body(state):
            kept, alive, l2v0, l2v1 = state
            best = jnp.max(jnp.maximum(l2v0, l2v1))
            bestv = jnp.full((16,), best)
            valid = best > (NEG / 2)

            def do_select(kept, l2v0, l2v1):
                g0 = plsc.all_reduce_ffs(l2v0 == bestv)
                g1 = plsc.all_reduce_ffs(l2v1 == bestv) + 16
                usev = g0 < 16
                gv = jnp.where(usev, g0, g1)
                cmaxg = plsc.load_gather(cmax, [gv * 16 + iota])
                cingv = plsc.all_reduce_ffs(cmaxg == bestv)
                cv = gv * 16 + cingv
                ptrv = plsc.load_gather(ptr, [cv])
                pv = cv * 16 + ptrv
                pay0 = plsc.load_gather(spay, [pv])
                ptr1 = ptrv + 1
                nk_raw = plsc.load_gather(skeys, [jnp.minimum(pv + 1, PAD_N - 1)])
                in_chunk = ptr1 < 16
                newm = jnp.where(in_chunk, nk_raw, negvec)
                plsc.store_scatter(ptr, [cv], ptr1, mask=lane0)
                plsc.store_scatter(cmax, [cv], newm, mask=lane0)
                newl2gv = bmax(jnp.where(iota == cingv, newm, cmaxg))
                g_in = jnp.where(usev, gv, gv - 16)
                lm = iota == g_in
                l2v0n = jnp.where(lm & usev, newl2gv, l2v0)
                l2v1n = jnp.where(lm & (~usev), newl2gv, l2v1)

                chidx = cv * 16 + iota
                chk = plsc.load_gather(skeys, [chidx])
                chp = plsc.load_gather(spay, [chidx])
                elig = (chk == bestv) & (iota >= ptrv)
                minpay = jnp.min(jnp.where(elig, chp, PAD_N))
                candv = jnp.full((16,), minpay)
                posm = plsc.all_reduce_ffs(elig & (chp == candv))
                plsc.store_scatter(spay, [cv * 16 + posm], pay0, mask=lane0)
                plsc.store_scatter(spay, [pv], candv, mask=lane0)

                by1 = plsc.load_gather(y1v, [candv])
                bx1 = plsc.load_gather(x1v, [candv])
                by2 = plsc.load_gather(y2v, [candv])
                bx2 = plsc.load_gather(x2v, [candv])
                bar = plsc.load_gather(arv, [candv])

                nk = (kept + 15) >> 4

                def iou_body(j, supv):
                    idxk = j * 16 + iota
                    kvy1 = plsc.load_gather(ky1, [idxk])
                    kvx1 = plsc.load_gather(kx1, [idxk])
                    kvy2 = plsc.load_gather(ky2, [idxk])
                    kvx2 = plsc.load_gather(kx2, [idxk])
                    kvar = (kvy2 - kvy1) * (kvx2 - kvx1)
                    ih = jnp.maximum(jnp.minimum(kvy2, by2) - jnp.maximum(kvy1, by1), 0.0)
                    iw = jnp.maximum(jnp.minimum(kvx2, bx2) - jnp.maximum(kvx1, bx1), 0.0)
                    inter = ih * iw
                    iou = inter / (bar + kvar - inter + 1e-9)
                    return supv | (iou > IOU_THR)

                supv = lax.fori_loop(0, nk, iou_body,
                                     jnp.zeros((16,), jnp.bool_))
                sup = jnp.any(supv)

                kslot = jnp.where(sup, KPAD - 1, kept) + zi
                plsc.store_scatter(ky1, [kslot], by1, mask=lane0)
                plsc.store_scatter(kx1, [kslot], bx1, mask=lane0)
                plsc.store_scatter(ky2, [kslot], by2, mask=lane0)
                plsc.store_scatter(kx2, [kslot], bx2, mask=lane0)
                obase = jnp.where(sup, OB_PAD - 4, kept * 4) + zi
                plsc.store_scatter(ob, [obase], by1, mask=lane0)
                plsc.store_scatter(ob, [obase + 1], bx1, mask=lane0)
                plsc.store_scatter(ob, [obase + 2], by2, mask=lane0)
                plsc.store_scatter(ob, [obase + 3], bx2, mask=lane0)
                return kept + jnp.where(sup, 0, 1), l2v0n, l2v1n

            kept2, l2v0b, l2v1b = lax.cond(
                valid, do_select, lambda k, a, bb: (k, a, bb), kept, l2v0, l2v1)
            alive2 = jnp.where(valid, alive, 0)
            return kept2, alive2, l2v0b, l2v1b

        kept_fin, _, _, _ = lax.while_loop(
            loop_cond, loop_body,
            (jnp.int32(0), jnp.int32(1), l2[0], l2[1]))

        nvv[...] = jnp.where(iota == 0, kept_fin, 0)
        pltpu.sync_copy(nvv.at[pl.ds(0, 8)], nv_hbm.at[b])
        pltpu.sync_copy(ob.at[pl.ds(0, MAX_OUT * 4)], ob_hbm.at[b])


_nms_call = functools.partial(
    pl.kernel,
    out_type=(jax.ShapeDtypeStruct((NUM_IMAGES, MAX_OUT * 4), jnp.float32),
              jax.ShapeDtypeStruct((NUM_IMAGES, 8), jnp.int32)),
    mesh=plsc.VectorSubcoreMesh(core_axis_name="c", subcore_axis_name="s"),
    compiler_params=pltpu.CompilerParams(
        needs_layout_passes=False, use_tc_tiling_on_sc=False),
    scratch_types=[
        pltpu.VMEM((N_BOXES, 2), jnp.float32),
        pltpu.VMEM((N_BOXES, 4), jnp.float32),
        pltpu.VMEM((16,), jnp.float32),
        pltpu.VMEM((PAD_N,), jnp.float32),
        pltpu.VMEM((PAD_N,), jnp.float32),
        pltpu.VMEM((PAD_N,), jnp.float32),
        pltpu.VMEM((PAD_N,), jnp.float32),
        pltpu.VMEM((PAD_N,), jnp.float32),
        pltpu.VMEM((PAD_N,), jnp.float32),
        pltpu.VMEM((PAD_N,), jnp.int32),
        pltpu.VMEM((512,), jnp.float32),
        pltpu.VMEM((N_CHUNKS,), jnp.int32),
        pltpu.VMEM((KPAD,), jnp.float32),
        pltpu.VMEM((KPAD,), jnp.float32),
        pltpu.VMEM((KPAD,), jnp.float32),
        pltpu.VMEM((KPAD,), jnp.float32),
        pltpu.VMEM((OB_PAD,), jnp.float32),
        pltpu.VMEM((16,), jnp.int32),
        pltpu.SemaphoreType.DMA,
    ],
)(_nms_body)


def kernel(rpn_probs, rpn_deltas, anchors):
    anchors16 = jnp.pad(anchors.reshape(-1), (4, 8))
    ob, nv = _nms_call(rpn_probs, rpn_deltas, anchors16)
    selected_boxes = ob.reshape(NUM_IMAGES * MAX_OUT, 4)
    selected_boxes_indices = jnp.repeat(
        jnp.arange(NUM_IMAGES, dtype=jnp.int32), MAX_OUT)
    num_valid = nv[:, 0]
    return selected_boxes, selected_boxes_indices, num_valid


# speedup vs baseline: 1.5574x; 1.5574x over previous
import functools

import jax
import jax.numpy as jnp
from jax import lax
from jax.experimental import pallas as pl
from jax.experimental.pallas import tpu as pltpu
from jax.experimental.pallas import tpu_sc as plsc

N_BOXES = 5000
PAD_N = 5120
N_CHUNKS = PAD_N // 16
N_GROUPS = N_CHUNKS // 16
MAX_OUT = 300
KPAD = 320
OB_PAD = 1216
NUM_IMAGES = 8
NEG = -1e30
IOU_THR = 0.7


def _nms_body(probs_hbm, deltas_hbm, anchors_hbm, ob_hbm, nv_hbm,
              pbuf, dbuf, anch, y1v, x1v, y2v, x2v, arv,
              skeys, spay, cmax, ptr,
              ky1, kx1, ky2, kx2, ob, nvv, sem):
    wid = lax.axis_index("s") * 2 + lax.axis_index("c")

    @pl.when(wid < NUM_IMAGES)
    def _():
        b = wid
        pltpu.sync_copy(anchors_hbm, anch)
        pltpu.sync_copy(probs_hbm.at[b], pbuf)
        pltpu.sync_copy(deltas_hbm.at[b], dbuf)

        iota = lax.iota(jnp.int32, 16)
        zi = jnp.zeros((16,), jnp.int32)
        lane0 = iota == 0
        negvec = jnp.full((16,), NEG, jnp.float32)
        zf = jnp.zeros((16,), jnp.float32)

        def bmax(v):
            return jnp.full((16,), jnp.max(v))

        xb = plsc.load_gather(anch, [zi + 4])
        yb = plsc.load_gather(anch, [zi + 5])
        wb = plsc.load_gather(anch, [zi + 6])
        hb = plsc.load_gather(anch, [zi + 7])

        for t in range(KPAD // 16):
            ky1[pl.ds(t * 16, 16)] = zf
            kx1[pl.ds(t * 16, 16)] = zf
            ky2[pl.ds(t * 16, 16)] = zf
            kx2[pl.ds(t * 16, 16)] = zf

        def decode_body(c, _):
            i = c * 16 + iota
            im = jnp.minimum(i, N_BOXES - 1)
            im2 = im * 2
            im4 = im * 4
            sc_ = plsc.load_gather(pbuf, [im2 + 1])
            sc_ = jnp.where(i < N_BOXES, sc_, negvec)
            xd = plsc.load_gather(dbuf, [im4])
            yd = plsc.load_gather(dbuf, [im4 + 1])
            wd = plsc.load_gather(dbuf, [im4 + 2])
            hd = plsc.load_gather(dbuf, [im4 + 3])
            y1 = jnp.minimum(jnp.maximum(xd * wb + xb, 0.0), 1.0)
            x1 = jnp.minimum(jnp.maximum(yd * hb + yb, 0.0), 1.0)
            y2 = jnp.minimum(jnp.maximum(jnp.exp(wd) * wb, 0.0), 1.0)
            x2 = jnp.minimum(jnp.maximum(jnp.exp(hd) * hb, 0.0), 1.0)
            plsc.store_scatter(y1v, [i], y1)
            plsc.store_scatter(x1v, [i], x1)
            plsc.store_scatter(y2v, [i], y2)
            plsc.store_scatter(x2v, [i], x2)
            plsc.store_scatter(arv, [i], (y2 - y1) * (x2 - x1))
            sk, sp_ = plsc.sort_key_val(sc_, i, descending=True)
            plsc.store_scatter(skeys, [i], sk)
            plsc.store_scatter(spay, [i], sp_)
            plsc.store_scatter(cmax, [jnp.full((16,), c)], sk, mask=lane0)
            return 0

        lax.fori_loop(0, N_CHUNKS, decode_body, 0)

        for t in range(N_GROUPS, 32):
            cmax[pl.ds(t * 16, 16)] = negvec
        for t in range(N_CHUNKS // 16):
            ptr[pl.ds(t * 16, 16)] = zi

        l2 = []
        for t in range(2):
            m = plsc.load_gather(cmax, [t * 256 + 16 * iota])
            for j in range(1, 16):
                m = jnp.maximum(m, plsc.load_gather(cmax, [t * 256 + 16 * iota + j]))
            l2.append(m)

        c4 = jnp.bitwise_and(iota, 3)
        vy10 = bmax(jnp.where(lane0, y1v[pl.ds(0, 16)], NEG))
        vx10 = bmax(jnp.where(lane0, x1v[pl.ds(0, 16)], NEG))
        vy20 = bmax(jnp.where(lane0, y2v[pl.ds(0, 16)], NEG))
        vx20 = bmax(jnp.where(lane0, x2v[pl.ds(0, 16)], NEG))
        pat = jnp.where(c4 == 0, vy10,
                        jnp.where(c4 == 1, vx10,
                                  jnp.where(c4 == 2, vy20, vx20)))
        for t in range(MAX_OUT * 4 // 16):
            ob[pl.ds(t * 16, 16)] = pat

        def loop_cond(state):
            kept, alive, _, _ = state
            return (kept < MAX_OUT) & (alive == 1)

        def loop_body(state):
            kept, alive, l2v0, l2v1 = state
            best = jnp.max(jnp.maximum(l2v0, l2v1))
            bestv = jnp.full((16,), best)
            valid = best > (NEG / 2)

            def do_select(kept, l2v0, l2v1):
                g0 = plsc.all_reduce_ffs(l2v0 == bestv)
                g1 = plsc.all_reduce_ffs(l2v1 == bestv) + 16
                usev = g0 < 16
                gv = jnp.where(usev, g0, g1)
                cmaxg = plsc.load_gather(cmax, [gv * 16 + iota])
                cingv = plsc.all_reduce_ffs(cmaxg == bestv)
                cv = gv * 16 + cingv
                ptrv = plsc.load_gather(ptr, [cv])
                pv = cv * 16 + ptrv
                pay0 = plsc.load_gather(spay, [pv])
                ptr1 = ptrv + 1
                nk_raw = plsc.load_gather(skeys, [jnp.minimum(pv + 1, PAD_N - 1)])
                in_chunk = ptr1 < 16
                newm = jnp.where(in_chunk, nk_raw, negvec)
                plsc.store_scatter(ptr, [cv], ptr1, mask=lane0)
                plsc.store_scatter(cmax, [cv], newm, mask=lane0)
                newl2gv = bmax(jnp.where(iota == cingv, newm, cmaxg))
                g_in = jnp.where(usev, gv, gv - 16)
                lm = iota == g_in
                l2v0n = jnp.where(lm & usev, newl2gv, l2v0)
                l2v1n = jnp.where(lm & (~usev), newl2gv, l2v1)

                chidx = cv * 16 + iota
                chk = plsc.load_gather(skeys, [chidx])
                chp = plsc.load_gather(spay, [chidx])
                elig = (chk == bestv) & (iota >= ptrv)
                minpay = jnp.min(jnp.where(elig, chp, PAD_N))
                candv = jnp.full((16,), minpay)
                posm = plsc.all_reduce_ffs(elig & (chp == candv))
                plsc.store_scatter(spay, [cv * 16 + posm], pay0, mask=lane0)
                plsc.store_scatter(spay, [pv], candv, mask=lane0)

                by1 = plsc.load_gather(y1v, [candv])
                bx1 = plsc.load_gather(x1v, [candv])
                by2 = plsc.load_gather(y2v, [candv])
                bx2 = plsc.load_gather(x2v, [candv])
                bar = plsc.load_gather(arv, [candv])

                nk = (kept + 15) >> 4

                def iou_body(j, supv):
                    idxk = j * 16 + iota
                    kvy1 = plsc.load_gather(ky1, [idxk])
                    kvx1 = plsc.load_gather(kx1, [idxk])
                    kvy2 = plsc.load_gather(ky2, [idxk])
                    kvx2 = plsc.load_gather(kx2, [idxk])
                    kvar = (kvy2 - kvy1) * (kvx2 - kvx1)
                    ih = jnp.maximum(jnp.minimum(kvy2, by2) - jnp.maximum(kvy1, by1), 0.0)
                    iw = jnp.maximum(jnp.minimum(kvx2, bx2) - jnp.maximum(kvx1, bx1), 0.0)
                    inter = ih * iw
                    iou = inter / (bar + kvar - inter + 1e-9)
                    return supv | (iou > IOU_THR)

                supv = lax.fori_loop(0, nk, iou_body,
                                     jnp.zeros((16,), jnp.bool_))
                sup = jnp.any(supv)

                kslot = jnp.where(sup, KPAD - 1, kept) + zi
                plsc.store_scatter(ky1, [kslot], by1, mask=lane0)
                plsc.store_scatter(kx1, [kslot], bx1, mask=lane0)
                plsc.store_scatter(ky2, [kslot], by2, mask=lane0)
                plsc.store_scatter(kx2, [kslot], bx2, mask=lane0)
                obase = jnp.where(sup, OB_PAD - 4, kept * 4) + zi
                plsc.store_scatter(ob, [obase], by1, mask=lane0)
                plsc.store_scatter(ob, [obase + 1], bx1, mask=lane0)
                plsc.store_scatter(ob, [obase + 2], by2, mask=lane0)
                plsc.store_scatter(ob, [obase + 3], bx2, mask=lane0)
                return kept + jnp.where(sup, 0, 1), l2v0n, l2v1n

            kept2, l2v0b, l2v1b = lax.cond(
                valid, do_select, lambda k, a, bb: (k, a, bb), kept, l2v0, l2v1)
            alive2 = jnp.where(valid, alive, 0)
            return kept2, alive2, l2v0b, l2v1b

        kept_fin, _, _, _ = lax.while_loop(
            loop_cond, loop_body,
            (jnp.int32(0), jnp.int32(1), l2[0], l2[1]))

        nvv[...] = jnp.where(iota == 0, kept_fin, 0)
        pltpu.sync_copy(nvv.at[pl.ds(0, 8)], nv_hbm.at[b])
        pltpu.sync_copy(ob.at[pl.ds(0, MAX_OUT * 4)], ob_hbm.at[b])


_nms_call = functools.partial(
    pl.kernel,
    out_type=(jax.ShapeDtypeStruct((NUM_IMAGES, MAX_OUT * 4), jnp.float32),
              jax.ShapeDtypeStruct((NUM_IMAGES, 8), jnp.int32)),
    mesh=plsc.VectorSubcoreMesh(core_axis_name="c", subcore_axis_name="s"),
    compiler_params=pltpu.CompilerParams(
        needs_layout_passes=False, use_tc_tiling_on_sc=False),
    scratch_types=[
        pltpu.VMEM((2 * N_BOXES,), jnp.float32),
        pltpu.VMEM((4 * N_BOXES,), jnp.float32),
        pltpu.VMEM((16,), jnp.float32),
        pltpu.VMEM((PAD_N,), jnp.float32),
        pltpu.VMEM((PAD_N,), jnp.float32),
        pltpu.VMEM((PAD_N,), jnp.float32),
        pltpu.VMEM((PAD_N,), jnp.float32),
        pltpu.VMEM((PAD_N,), jnp.float32),
        pltpu.VMEM((PAD_N,), jnp.float32),
        pltpu.VMEM((PAD_N,), jnp.int32),
        pltpu.VMEM((512,), jnp.float32),
        pltpu.VMEM((N_CHUNKS,), jnp.int32),
        pltpu.VMEM((KPAD,), jnp.float32),
        pltpu.VMEM((KPAD,), jnp.float32),
        pltpu.VMEM((KPAD,), jnp.float32),
        pltpu.VMEM((KPAD,), jnp.float32),
        pltpu.VMEM((OB_PAD,), jnp.float32),
        pltpu.VMEM((16,), jnp.int32),
        pltpu.SemaphoreType.DMA,
    ],
)(_nms_body)


def kernel(rpn_probs, rpn_deltas, anchors):
    anchors16 = jnp.pad(anchors.reshape(-1), (4, 8))
    ob, nv = _nms_call(rpn_probs.reshape(NUM_IMAGES, -1),
                       rpn_deltas.reshape(NUM_IMAGES, -1), anchors16)
    selected_boxes = ob.reshape(NUM_IMAGES * MAX_OUT, 4)
    selected_boxes_indices = jnp.repeat(
        jnp.arange(NUM_IMAGES, dtype=jnp.int32), MAX_OUT)
    num_valid = nv[:, 0]
    return selected_boxes, selected_boxes_indices, num_valid


# speedup vs baseline: 1.9592x; 1.2580x over previous
import functools

import jax
import jax.numpy as jnp
from jax import lax
from jax.experimental import pallas as pl
from jax.experimental.pallas import tpu as pltpu
from jax.experimental.pallas import tpu_sc as plsc

N_BOXES = 5000
PAD_N = 5120
N_CHUNKS = PAD_N // 16
N_GROUPS = N_CHUNKS // 16
MAX_OUT = 300
KPAD = 320
OB_PAD = 1216
NUM_IMAGES = 8
NEG = -1e30
IOU_THR = 0.7

P_SC = 0
P_Y1 = PAD_N
P_X1 = 2 * PAD_N
P_Y2 = 3 * PAD_N
P_X2 = 4 * PAD_N
P_AR = 5 * PAD_N


def _decode_body(anchors_ref, dt_ref, sc_ref, out_ref):
    xb = anchors_ref[0, 0]
    yb = anchors_ref[0, 1]
    wb = anchors_ref[0, 2]
    hb = anchors_ref[0, 3]
    xd = dt_ref[0]
    yd = dt_ref[1]
    wd = dt_ref[2]
    hd = dt_ref[3]
    y1 = jnp.minimum(jnp.maximum(xd * wb + xb, 0.0), 1.0)
    x1 = jnp.minimum(jnp.maximum(yd * hb + yb, 0.0), 1.0)
    y2 = jnp.minimum(jnp.maximum(jnp.exp(wd) * wb, 0.0), 1.0)
    x2 = jnp.minimum(jnp.maximum(jnp.exp(hd) * hb, 0.0), 1.0)
    out_ref[:, P_SC:P_SC + PAD_N] = sc_ref[...]
    out_ref[:, P_Y1:P_Y1 + PAD_N] = y1
    out_ref[:, P_X1:P_X1 + PAD_N] = x1
    out_ref[:, P_Y2:P_Y2 + PAD_N] = y2
    out_ref[:, P_X2:P_X2 + PAD_N] = x2
    out_ref[:, P_AR:P_AR + PAD_N] = (y2 - y1) * (x2 - x1)


_decode_call = pl.pallas_call(
    _decode_body,
    out_shape=jax.ShapeDtypeStruct((NUM_IMAGES, 6 * PAD_N), jnp.float32),
    in_specs=[
        pl.BlockSpec(memory_space=pltpu.SMEM),
        pl.BlockSpec(memory_space=pltpu.VMEM),
        pl.BlockSpec(memory_space=pltpu.VMEM),
    ],
)


def _nms_body(planes_hbm, ob_hbm, nv_hbm,
              buf, skeys, spay, cmax, ptr,
              ky1, kx1, ky2, kx2, ob, nvv, sem):
    wid = lax.axis_index("s") * 2 + lax.axis_index("c")

    @pl.when(wid < NUM_IMAGES)
    def _():
        b = wid
        pltpu.sync_copy(planes_hbm.at[b], buf)

        iota = lax.iota(jnp.int32, 16)
        zi = jnp.zeros((16,), jnp.int32)
        lane0 = iota == 0
        negvec = jnp.full((16,), NEG, jnp.float32)
        zf = jnp.zeros((16,), jnp.float32)

        def bmax(v):
            return jnp.full((16,), jnp.max(v))

        for t in range(KPAD // 16):
            ky1[pl.ds(t * 16, 16)] = zf
            kx1[pl.ds(t * 16, 16)] = zf
            ky2[pl.ds(t * 16, 16)] = zf
            kx2[pl.ds(t * 16, 16)] = zf

        for c in range(N_CHUNKS):
            k = buf[pl.ds(P_SC + c * 16, 16)]
            sk, sp_ = plsc.sort_key_val(k, c * 16 + iota, descending=True)
            skeys[pl.ds(c * 16, 16)] = sk
            spay[pl.ds(c * 16, 16)] = sp_

        for g in range(N_GROUPS):
            cm = plsc.load_gather(skeys, [g * 256 + 16 * iota])
            cmax[pl.ds(g * 16, 16)] = cm
        for t in range(N_GROUPS, 32):
            cmax[pl.ds(t * 16, 16)] = negvec
        for t in range(N_CHUNKS // 16):
            ptr[pl.ds(t * 16, 16)] = zi

        l2 = []
        for t in range(2):
            m = plsc.load_gather(cmax, [t * 256 + 16 * iota])
            for j in range(1, 16):
                m = jnp.maximum(m, plsc.load_gather(cmax, [t * 256 + 16 * iota + j]))
            l2.append(m)

        c4 = jnp.bitwise_and(iota, 3)
        vy10 = plsc.load_gather(buf, [zi + P_Y1])
        vx10 = plsc.load_gather(buf, [zi + P_X1])
        vy20 = plsc.load_gather(buf, [zi + P_Y2])
        vx20 = plsc.load_gather(buf, [zi + P_X2])
        pat = jnp.where(c4 == 0, vy10,
                        jnp.where(c4 == 1, vx10,
                                  jnp.where(c4 == 2, vy20, vx20)))
        for t in range(MAX_OUT * 4 // 16):
            ob[pl.ds(t * 16, 16)] = pat

        def loop_cond(state):
            kept, alive, _, _ = state
            return (kept < MAX_OUT) & (alive == 1)

        def loop_body(state):
            kept, alive, l2v0, l2v1 = state
            best = jnp.max(jnp.maximum(l2v0, l2v1))
            bestv = jnp.full((16,), best)
            valid = best > (NEG / 2)

            def do_select(kept, l2v0, l2v1):
                g0 = plsc.all_reduce_ffs(l2v0 == bestv)
                g1 = plsc.all_reduce_ffs(l2v1 == bestv) + 16
                usev = g0 < 16
                gv = jnp.where(usev, g0, g1)
                cmaxg = plsc.load_gather(cmax, [gv * 16 + iota])
                cingv = plsc.all_reduce_ffs(cmaxg == bestv)
                cv = gv * 16 + cingv
                ptrv = plsc.load_gather(ptr, [cv])
                pv = cv * 16 + ptrv
                pay0 = plsc.load_gather(spay, [pv])
                ptr1 = ptrv + 1
                nk_raw = plsc.load_gather(skeys, [jnp.minimum(pv + 1, PAD_N - 1)])
                in_chunk = ptr1 < 16
                newm = jnp.where(in_chunk, nk_raw, negvec)
                plsc.store_scatter(ptr, [cv], ptr1, mask=lane0)
                plsc.store_scatter(cmax, [cv], newm, mask=lane0)

                chidx = cv * 16 + iota
                chk = plsc.load_gather(skeys, [chidx])
                chp = plsc.load_gather(spay, [chidx])
                elig = (chk == bestv) & (iota >= ptrv)
                minpay = jnp.min(jnp.where(elig, chp, PAD_N))
                candv = jnp.full((16,), minpay)
                posm = plsc.all_reduce_ffs(elig & (chp == candv))
                plsc.store_scatter(spay, [cv * 16 + posm], pay0, mask=lane0)
                plsc.store_scatter(spay, [pv], candv, mask=lane0)

                by1 = plsc.load_gather(buf, [candv + P_Y1])
                bx1 = plsc.load_gather(buf, [candv + P_X1])
                by2 = plsc.load_gather(buf, [candv + P_Y2])
                bx2 = plsc.load_gather(buf, [candv + P_X2])
                bar = (by2 - by1) * (bx2 - bx1)

                nk = (kept + 15) >> 4

                def iou_body(j, supv):
                    idxk = j * 16 + iota
                    kvy1 = plsc.load_gather(ky1, [idxk])
                    kvx1 = plsc.load_gather(kx1, [idxk])
                    kvy2 = plsc.load_gather(ky2, [idxk])
                    kvx2 = plsc.load_gather(kx2, [idxk])
                    kvar = (kvy2 - kvy1) * (kvx2 - kvx1)
                    ih = jnp.maximum(jnp.minimum(kvy2, by2) - jnp.maximum(kvy1, by1), 0.0)
                    iw = jnp.maximum(jnp.minimum(kvx2, bx2) - jnp.maximum(kvx1, bx1), 0.0)
                    inter = ih * iw
                    iou = inter / (bar + kvar - inter + 1e-9)
                    return supv | (iou > IOU_THR)

                supv = lax.fori_loop(0, nk, iou_body,
                                     jnp.zeros((16,), jnp.bool_))
                newl2gv = bmax(jnp.where(iota == cingv, newm, cmaxg))
                sup = jnp.any(supv)
                g_in = jnp.where(usev, gv, gv - 16)
                lm = iota == g_in
                l2v0n = jnp.where(lm & usev, newl2gv, l2v0)
                l2v1n = jnp.where(lm & (~usev), newl2gv, l2v1)

                kslot = jnp.where(sup, KPAD - 1, kept) + zi
                plsc.store_scatter(ky1, [kslot], by1, mask=lane0)
                plsc.store_scatter(kx1, [kslot], bx1, mask=lane0)
                plsc.store_scatter(ky2, [kslot], by2, mask=lane0)
                plsc.store_scatter(kx2, [kslot], bx2, mask=lane0)
                obase = jnp.where(sup, OB_PAD - 4, kept * 4) + zi
                plsc.store_scatter(ob, [obase], by1, mask=lane0)
                plsc.store_scatter(ob, [obase + 1], bx1, mask=lane0)
                plsc.store_scatter(ob, [obase + 2], by2, mask=lane0)
                plsc.store_scatter(ob, [obase + 3], bx2, mask=lane0)
                return kept + jnp.where(sup, 0, 1), l2v0n, l2v1n

            kept2, l2v0b, l2v1b = lax.cond(
                valid, do_select, lambda k, a, bb: (k, a, bb), kept, l2v0, l2v1)
            alive2 = jnp.where(valid, alive, 0)
            return kept2, alive2, l2v0b, l2v1b

        kept_fin, _, _, _ = lax.while_loop(
            loop_cond, loop_body,
            (jnp.int32(0), jnp.int32(1), l2[0], l2[1]))

        nvv[...] = jnp.where(iota == 0, kept_fin, 0)
        pltpu.sync_copy(nvv.at[pl.ds(0, 8)], nv_hbm.at[b])
        pltpu.sync_copy(ob.at[pl.ds(0, MAX_OUT * 4)], ob_hbm.at[b])


_nms_call = functools.partial(
    pl.kernel,
    out_type=(jax.ShapeDtypeStruct((NUM_IMAGES, MAX_OUT * 4), jnp.float32),
              jax.ShapeDtypeStruct((NUM_IMAGES, 8), jnp.int32)),
    mesh=plsc.VectorSubcoreMesh(core_axis_name="c", subcore_axis_name="s"),
    compiler_params=pltpu.CompilerParams(
        needs_layout_passes=False, use_tc_tiling_on_sc=False),
    scratch_types=[
        pltpu.VMEM((6 * PAD_N,), jnp.float32),
        pltpu.VMEM((PAD_N,), jnp.float32),
        pltpu.VMEM((PAD_N,), jnp.int32),
        pltpu.VMEM((512,), jnp.float32),
        pltpu.VMEM((N_CHUNKS,), jnp.int32),
        pltpu.VMEM((KPAD,), jnp.float32),
        pltpu.VMEM((KPAD,), jnp.float32),
        pltpu.VMEM((KPAD,), jnp.float32),
        pltpu.VMEM((KPAD,), jnp.float32),
        pltpu.VMEM((OB_PAD,), jnp.float32),
        pltpu.VMEM((16,), jnp.int32),
        pltpu.SemaphoreType.DMA,
    ],
)(_nms_body)


def kernel(rpn_probs, rpn_deltas, anchors):
    dp = jnp.pad(rpn_deltas, ((0, 0), (0, PAD_N - N_BOXES), (0, 0)))
    dt = jnp.transpose(dp, (2, 0, 1))
    sp = jnp.pad(rpn_probs[:, :, 1], ((0, 0), (0, PAD_N - N_BOXES)),
                 constant_values=NEG)
    planes = _decode_call(anchors, dt, sp)
    ob, nv = _nms_call(planes)
    selected_boxes = ob.reshape(NUM_IMAGES * MAX_OUT, 4)
    selected_boxes_indices = jnp.repeat(
        jnp.arange(NUM_IMAGES, dtype=jnp.int32), MAX_OUT)
    num_valid = nv[:, 0]
    return selected_boxes, selected_boxes_indices, num_valid


# speedup vs baseline: 2.0111x; 1.0265x over previous
import functools

import jax
import jax.numpy as jnp
from jax import lax
from jax.experimental import pallas as pl
from jax.experimental.pallas import tpu as pltpu
from jax.experimental.pallas import tpu_sc as plsc

N_BOXES = 5000
PAD_N = 5120
N_CHUNKS = PAD_N // 16
N_GROUPS = N_CHUNKS // 16
MAX_OUT = 300
KPAD = 320
OB_PAD = 1216
NUM_IMAGES = 8
NEG = -1e30
IOU_THR = 0.7

P_SC = 0
P_Y1 = PAD_N
P_X1 = 2 * PAD_N
P_Y2 = 3 * PAD_N
P_X2 = 4 * PAD_N
P_AR = 5 * PAD_N


def _decode_body(anchors_ref, dt_ref, sc_ref, out_ref):
    xb = anchors_ref[0, 0]
    yb = anchors_ref[0, 1]
    wb = anchors_ref[0, 2]
    hb = anchors_ref[0, 3]
    xd = dt_ref[0]
    yd = dt_ref[1]
    wd = dt_ref[2]
    hd = dt_ref[3]
    y1 = jnp.minimum(jnp.maximum(xd * wb + xb, 0.0), 1.0)
    x1 = jnp.minimum(jnp.maximum(yd * hb + yb, 0.0), 1.0)
    y2 = jnp.minimum(jnp.maximum(jnp.exp(wd) * wb, 0.0), 1.0)
    x2 = jnp.minimum(jnp.maximum(jnp.exp(hd) * hb, 0.0), 1.0)
    out_ref[:, P_SC:P_SC + PAD_N] = sc_ref[...]
    out_ref[:, P_Y1:P_Y1 + PAD_N] = y1
    out_ref[:, P_X1:P_X1 + PAD_N] = x1
    out_ref[:, P_Y2:P_Y2 + PAD_N] = y2
    out_ref[:, P_X2:P_X2 + PAD_N] = x2
    out_ref[:, P_AR:P_AR + PAD_N] = (y2 - y1) * (x2 - x1)


_decode_call = pl.pallas_call(
    _decode_body,
    out_shape=jax.ShapeDtypeStruct((NUM_IMAGES, 6 * PAD_N), jnp.float32),
    in_specs=[
        pl.BlockSpec(memory_space=pltpu.SMEM),
        pl.BlockSpec(memory_space=pltpu.VMEM),
        pl.BlockSpec(memory_space=pltpu.VMEM),
    ],
)


def _nms_body(planes_hbm, ob_hbm, nv_hbm,
              buf, skeys, spay, cmax, ptr,
              ky1, kx1, ky2, kx2, ob, nvv, sem):
    wid = lax.axis_index("s") * 2 + lax.axis_index("c")

    @pl.when(wid < NUM_IMAGES)
    def _():
        b = wid
        pltpu.sync_copy(planes_hbm.at[b], buf)

        iota = lax.iota(jnp.int32, 16)
        zi = jnp.zeros((16,), jnp.int32)
        lane0 = iota == 0
        negvec = jnp.full((16,), NEG, jnp.float32)
        zf = jnp.zeros((16,), jnp.float32)

        def bmax(v):
            return jnp.full((16,), jnp.max(v))

        for t in range(KPAD // 16):
            ky1[pl.ds(t * 16, 16)] = zf
            kx1[pl.ds(t * 16, 16)] = zf
            ky2[pl.ds(t * 16, 16)] = zf
            kx2[pl.ds(t * 16, 16)] = zf

        for c in range(N_CHUNKS):
            k = buf[pl.ds(P_SC + c * 16, 16)]
            sk, sp_ = plsc.sort_key_val(k, c * 16 + iota, descending=True)
            skeys[pl.ds(c * 16, 16)] = sk
            spay[pl.ds(c * 16, 16)] = sp_

        for g in range(N_GROUPS):
            cm = plsc.load_gather(skeys, [g * 256 + 16 * iota])
            cmax[pl.ds(g * 16, 16)] = cm
        for t in range(N_GROUPS, 32):
            cmax[pl.ds(t * 16, 16)] = negvec
        for t in range(N_CHUNKS // 16):
            ptr[pl.ds(t * 16, 16)] = zi

        l2 = []
        for t in range(2):
            m = plsc.load_gather(cmax, [t * 256 + 16 * iota])
            for j in range(1, 16):
                m = jnp.maximum(m, plsc.load_gather(cmax, [t * 256 + 16 * iota + j]))
            l2.append(m)

        c4 = jnp.bitwise_and(iota, 3)
        vy10 = plsc.load_gather(buf, [zi + P_Y1])
        vx10 = plsc.load_gather(buf, [zi + P_X1])
        vy20 = plsc.load_gather(buf, [zi + P_Y2])
        vx20 = plsc.load_gather(buf, [zi + P_X2])
        pat = jnp.where(c4 == 0, vy10,
                        jnp.where(c4 == 1, vx10,
                                  jnp.where(c4 == 2, vy20, vx20)))
        for t in range(MAX_OUT * 4 // 16):
            ob[pl.ds(t * 16, 16)] = pat

        def loop_cond(state):
            kept, alive, _, _ = state
            return (kept < MAX_OUT) & (alive == 1)

        def loop_body(state):
            kept, alive, l2v0, l2v1 = state
            best = jnp.max(jnp.maximum(l2v0, l2v1))
            bestv = jnp.full((16,), best)
            valid = best > (NEG / 2)

            def do_select(kept, l2v0, l2v1):
                g0 = plsc.all_reduce_ffs(l2v0 == bestv)
                g1 = plsc.all_reduce_ffs(l2v1 == bestv) + 16
                usev = g0 < 16
                gv = jnp.where(usev, g0, g1)
                cmaxg = plsc.load_gather(cmax, [gv * 16 + iota])
                cingv = plsc.all_reduce_ffs(cmaxg == bestv)
                cv = gv * 16 + cingv
                ptrv = plsc.load_gather(ptr, [cv])
                pv = cv * 16 + ptrv
                pay0 = plsc.load_gather(spay, [pv])
                ptr1 = ptrv + 1
                nk_raw = plsc.load_gather(skeys, [jnp.minimum(pv + 1, PAD_N - 1)])
                in_chunk = ptr1 < 16
                newm = jnp.where(in_chunk, nk_raw, negvec)
                plsc.store_scatter(ptr, [cv], ptr1, mask=lane0)
                plsc.store_scatter(cmax, [cv], newm, mask=lane0)
                newl2gv = bmax(jnp.where(iota == cingv, newm, cmaxg))
                g_in = jnp.where(usev, gv, gv - 16)
                lm = iota == g_in
                l2v0n = jnp.where(lm & usev, newl2gv, l2v0)
                l2v1n = jnp.where(lm & (~usev), newl2gv, l2v1)

                chidx = cv * 16 + iota
                chk = plsc.load_gather(skeys, [chidx])
                chp = plsc.load_gather(spay, [chidx])
                elig = (chk == bestv) & (iota >= ptrv)
                minpay = jnp.min(jnp.where(elig, chp, PAD_N))
                candv = jnp.full((16,), minpay)
                posm = plsc.all_reduce_ffs(elig & (chp == candv))
                plsc.store_scatter(spay, [cv * 16 + posm], pay0, mask=lane0)
                plsc.store_scatter(spay, [pv], candv, mask=lane0)

                by1 = plsc.load_gather(buf, [candv + P_Y1])
                bx1 = plsc.load_gather(buf, [candv + P_X1])
                by2 = plsc.load_gather(buf, [candv + P_Y2])
                bx2 = plsc.load_gather(buf, [candv + P_X2])
                bar = (by2 - by1) * (bx2 - bx1)

                nk = (kept + 15) >> 4

                def iou_body(j, supv):
                    idxk = j * 16 + iota
                    kvy1 = plsc.load_gather(ky1, [idxk])
                    kvx1 = plsc.load_gather(kx1, [idxk])
                    kvy2 = plsc.load_gather(ky2, [idxk])
                    kvx2 = plsc.load_gather(kx2, [idxk])
                    kvar = (kvy2 - kvy1) * (kvx2 - kvx1)
                    ih = jnp.maximum(jnp.minimum(kvy2, by2) - jnp.maximum(kvy1, by1), 0.0)
                    iw = jnp.maximum(jnp.minimum(kvx2, bx2) - jnp.maximum(kvx1, bx1), 0.0)
                    inter = ih * iw
                    iou = inter / (bar + kvar - inter + 1e-9)
                    return supv | (iou > IOU_THR)

                supv = lax.fori_loop(0, nk, iou_body,
                                     jnp.zeros((16,), jnp.bool_))
                sup = jnp.any(supv)

                kslot = jnp.where(sup, KPAD - 1, kept) + zi
                plsc.store_scatter(ky1, [kslot], by1, mask=lane0)
                plsc.store_scatter(kx1, [kslot], bx1, mask=lane0)
                plsc.store_scatter(ky2, [kslot], by2, mask=lane0)
                plsc.store_scatter(kx2, [kslot], bx2, mask=lane0)
                obase = jnp.where(sup, OB_PAD - 4, kept * 4) + zi
                plsc.store_scatter(ob, [obase], by1, mask=lane0)
                plsc.store_scatter(ob, [obase + 1], bx1, mask=lane0)
                plsc.store_scatter(ob, [obase + 2], by2, mask=lane0)
                plsc.store_scatter(ob, [obase + 3], bx2, mask=lane0)
                return kept + jnp.where(sup, 0, 1), l2v0n, l2v1n

            kept2, l2v0b, l2v1b = lax.cond(
                valid, do_select, lambda k, a, bb: (k, a, bb), kept, l2v0, l2v1)
            alive2 = jnp.where(valid, alive, 0)
            return kept2, alive2, l2v0b, l2v1b

        kept_fin, _, _, _ = lax.while_loop(
            loop_cond, loop_body,
            (jnp.int32(0), jnp.int32(1), l2[0], l2[1]))

        nvv[...] = jnp.where(iota == 0, kept_fin, 0)
        pltpu.sync_copy(nvv.at[pl.ds(0, 8)], nv_hbm.at[b])
        pltpu.sync_copy(ob.at[pl.ds(0, MAX_OUT * 4)], ob_hbm.at[b])


_nms_call = functools.partial(
    pl.kernel,
    out_type=(jax.ShapeDtypeStruct((NUM_IMAGES, MAX_OUT * 4), jnp.float32),
              jax.ShapeDtypeStruct((NUM_IMAGES, 8), jnp.int32)),
    mesh=plsc.VectorSubcoreMesh(core_axis_name="c", subcore_axis_name="s"),
    compiler_params=pltpu.CompilerParams(
        needs_layout_passes=False, use_tc_tiling_on_sc=False),
    scratch_types=[
        pltpu.VMEM((6 * PAD_N,), jnp.float32),
        pltpu.VMEM((PAD_N,), jnp.float32),
        pltpu.VMEM((PAD_N,), jnp.int32),
        pltpu.VMEM((512,), jnp.float32),
        pltpu.VMEM((N_CHUNKS,), jnp.int32),
        pltpu.VMEM((KPAD,), jnp.float32),
        pltpu.VMEM((KPAD,), jnp.float32),
        pltpu.VMEM((KPAD,), jnp.float32),
        pltpu.VMEM((KPAD,), jnp.float32),
        pltpu.VMEM((OB_PAD,), jnp.float32),
        pltpu.VMEM((16,), jnp.int32),
        pltpu.SemaphoreType.DMA,
    ],
)(_nms_body)


def kernel(rpn_probs, rpn_deltas, anchors):
    dp = jnp.pad(rpn_deltas, ((0, 0), (0, PAD_N - N_BOXES), (0, 0)))
    dt = jnp.transpose(dp, (2, 0, 1))
    sp = jnp.pad(rpn_probs[:, :, 1], ((0, 0), (0, PAD_N - N_BOXES)),
                 constant_values=NEG)
    planes = _decode_call(anchors, dt, sp)
    ob, nv = _nms_call(planes)
    selected_boxes = ob.reshape(NUM_IMAGES * MAX_OUT, 4)
    selected_boxes_indices = jnp.repeat(
        jnp.arange(NUM_IMAGES, dtype=jnp.int32), MAX_OUT)
    num_valid = nv[:, 0]
    return selected_boxes, selected_boxes_indices, num_valid


# speedup vs baseline: 2.0325x; 1.0106x over previous
import functools

import jax
import jax.numpy as jnp
from jax import lax
from jax.experimental import pallas as pl
from jax.experimental.pallas import tpu as pltpu
from jax.experimental.pallas import tpu_sc as plsc

N_BOXES = 5000
PAD_N = 5120
N_CHUNKS = PAD_N // 16
N_GROUPS = N_CHUNKS // 16
MAX_OUT = 300
KPAD = 320
OB_PAD = 1216
NUM_IMAGES = 8
NEG = -1e30
IOU_THR = 0.7

P_SC = 0
P_Y1 = PAD_N
P_X1 = 2 * PAD_N
P_Y2 = 3 * PAD_N
P_X2 = 4 * PAD_N
P_AR = 5 * PAD_N


def _decode_body(anchors_ref, dt_ref, sc_ref, out_ref):
    xb = anchors_ref[0, 0]
    yb = anchors_ref[0, 1]
    wb = anchors_ref[0, 2]
    hb = anchors_ref[0, 3]
    xd = dt_ref[0]
    yd = dt_ref[1]
    wd = dt_ref[2]
    hd = dt_ref[3]
    y1 = jnp.minimum(jnp.maximum(xd * wb + xb, 0.0), 1.0)
    x1 = jnp.minimum(jnp.maximum(yd * hb + yb, 0.0), 1.0)
    y2 = jnp.minimum(jnp.maximum(jnp.exp(wd) * wb, 0.0), 1.0)
    x2 = jnp.minimum(jnp.maximum(jnp.exp(hd) * hb, 0.0), 1.0)
    out_ref[:, P_SC:P_SC + N_BOXES] = sc_ref[...]
    out_ref[:, P_SC + N_BOXES:P_SC + PAD_N] = jnp.full(
        (NUM_IMAGES, PAD_N - N_BOXES), NEG, jnp.float32)
    out_ref[:, P_Y1:P_Y1 + N_BOXES] = y1
    out_ref[:, P_X1:P_X1 + N_BOXES] = x1
    out_ref[:, P_Y2:P_Y2 + N_BOXES] = y2
    out_ref[:, P_X2:P_X2 + N_BOXES] = x2
    out_ref[:, P_AR:P_AR + N_BOXES] = (y2 - y1) * (x2 - x1)
    for base in (P_Y1, P_X1, P_Y2, P_X2, P_AR):
        out_ref[:, base + N_BOXES:base + PAD_N] = jnp.zeros(
            (NUM_IMAGES, PAD_N - N_BOXES), jnp.float32)


_decode_call = pl.pallas_call(
    _decode_body,
    out_shape=jax.ShapeDtypeStruct((NUM_IMAGES, 6 * PAD_N), jnp.float32),
    in_specs=[
        pl.BlockSpec(memory_space=pltpu.SMEM),
        pl.BlockSpec(memory_space=pltpu.VMEM),
        pl.BlockSpec(memory_space=pltpu.VMEM),
    ],
)


def _nms_body(planes_hbm, ob_hbm, nv_hbm,
              buf, skeys, spay, cmax, ptr,
              ky1, kx1, ky2, kx2, ob, nvv, sem):
    wid = lax.axis_index("s") * 2 + lax.axis_index("c")

    @pl.when(wid < NUM_IMAGES)
    def _():
        b = wid
        pltpu.sync_copy(planes_hbm.at[b], buf)

        iota = lax.iota(jnp.int32, 16)
        zi = jnp.zeros((16,), jnp.int32)
        lane0 = iota == 0
        negvec = jnp.full((16,), NEG, jnp.float32)
        zf = jnp.zeros((16,), jnp.float32)

        def bmax(v):
            return jnp.full((16,), jnp.max(v))

        for t in range(KPAD // 16):
            ky1[pl.ds(t * 16, 16)] = zf
            kx1[pl.ds(t * 16, 16)] = zf
            ky2[pl.ds(t * 16, 16)] = zf
            kx2[pl.ds(t * 16, 16)] = zf

        for c in range(N_CHUNKS):
            k = buf[pl.ds(P_SC + c * 16, 16)]
            sk, sp_ = plsc.sort_key_val(k, c * 16 + iota, descending=True)
            skeys[pl.ds(c * 16, 16)] = sk
            spay[pl.ds(c * 16, 16)] = sp_

        for g in range(N_GROUPS):
            cm = plsc.load_gather(skeys, [g * 256 + 16 * iota])
            cmax[pl.ds(g * 16, 16)] = cm
        for t in range(N_GROUPS, 32):
            cmax[pl.ds(t * 16, 16)] = negvec
        for t in range(N_CHUNKS // 16):
            ptr[pl.ds(t * 16, 16)] = zi

        l2 = []
        for t in range(2):
            m = plsc.load_gather(cmax, [t * 256 + 16 * iota])
            for j in range(1, 16):
                m = jnp.maximum(m, plsc.load_gather(cmax, [t * 256 + 16 * iota + j]))
            l2.append(m)

        c4 = jnp.bitwise_and(iota, 3)
        vy10 = plsc.load_gather(buf, [zi + P_Y1])
        vx10 = plsc.load_gather(buf, [zi + P_X1])
        vy20 = plsc.load_gather(buf, [zi + P_Y2])
        vx20 = plsc.load_gather(buf, [zi + P_X2])
        pat = jnp.where(c4 == 0, vy10,
                        jnp.where(c4 == 1, vx10,
                                  jnp.where(c4 == 2, vy20, vx20)))
        for t in range(MAX_OUT * 4 // 16):
            ob[pl.ds(t * 16, 16)] = pat

        def loop_cond(state):
            kept, alive, _, _ = state
            return (kept < MAX_OUT) & (alive == 1)

        def loop_body(state):
            kept, alive, l2v0, l2v1 = state
            best = jnp.max(jnp.maximum(l2v0, l2v1))
            bestv = jnp.full((16,), best)
            valid = best > (NEG / 2)

            def do_select(kept, l2v0, l2v1):
                g0 = plsc.all_reduce_ffs(l2v0 == bestv)
                g1 = plsc.all_reduce_ffs(l2v1 == bestv) + 16
                usev = g0 < 16
                gv = jnp.where(usev, g0, g1)
                cmaxg = plsc.load_gather(cmax, [gv * 16 + iota])
                cingv = plsc.all_reduce_ffs(cmaxg == bestv)
                cv = gv * 16 + cingv
                ptrv = plsc.load_gather(ptr, [cv])
                pv = cv * 16 + ptrv
                pay0 = plsc.load_gather(spay, [pv])
                ptr1 = ptrv + 1
                nk_raw = plsc.load_gather(skeys, [jnp.minimum(pv + 1, PAD_N - 1)])
                in_chunk = ptr1 < 16
                newm = jnp.where(in_chunk, nk_raw, negvec)
                plsc.store_scatter(ptr, [cv], ptr1, mask=lane0)
                plsc.store_scatter(cmax, [cv], newm, mask=lane0)
                newl2gv = bmax(jnp.where(iota == cingv, newm, cmaxg))
                g_in = jnp.where(usev, gv, gv - 16)
                lm = iota == g_in
                l2v0n = jnp.where(lm & usev, newl2gv, l2v0)
                l2v1n = jnp.where(lm & (~usev), newl2gv, l2v1)

                chidx = cv * 16 + iota
                chk = plsc.load_gather(skeys, [chidx])
                chp = plsc.load_gather(spay, [chidx])
                elig = (chk == bestv) & (iota >= ptrv)
                minpay = jnp.min(jnp.where(elig, chp, PAD_N))
                candv = jnp.full((16,), minpay)
                posm = plsc.all_reduce_ffs(elig & (chp == candv))
                plsc.store_scatter(spay, [cv * 16 + posm], pay0, mask=lane0)
                plsc.store_scatter(spay, [pv], candv, mask=lane0)

                by1 = plsc.load_gather(buf, [candv + P_Y1])
                bx1 = plsc.load_gather(buf, [candv + P_X1])
                by2 = plsc.load_gather(buf, [candv + P_Y2])
                bx2 = plsc.load_gather(buf, [candv + P_X2])
                bar = (by2 - by1) * (bx2 - bx1)

                nk = (kept + 15) >> 4

                def iou_body(j, supv):
                    idxk = j * 16 + iota
                    kvy1 = plsc.load_gather(ky1, [idxk])
                    kvx1 = plsc.load_gather(kx1, [idxk])
                    kvy2 = plsc.load_gather(ky2, [idxk])
                    kvx2 = plsc.load_gather(kx2, [idxk])
                    kvar = (kvy2 - kvy1) * (kvx2 - kvx1)
                    ih = jnp.maximum(jnp.minimum(kvy2, by2) - jnp.maximum(kvy1, by1), 0.0)
                    iw = jnp.maximum(jnp.minimum(kvx2, bx2) - jnp.maximum(kvx1, bx1), 0.0)
                    inter = ih * iw
                    iou = inter / (bar + kvar - inter + 1e-9)
                    return supv | (iou > IOU_THR)

                supv = lax.fori_loop(0, nk, iou_body,
                                     jnp.zeros((16,), jnp.bool_))
                sup = jnp.any(supv)

                kslot = jnp.where(sup, KPAD - 1, kept) + zi
                plsc.store_scatter(ky1, [kslot], by1, mask=lane0)
                plsc.store_scatter(kx1, [kslot], bx1, mask=lane0)
                plsc.store_scatter(ky2, [kslot], by2, mask=lane0)
                plsc.store_scatter(kx2, [kslot], bx2, mask=lane0)
                obase = jnp.where(sup, OB_PAD - 4, kept * 4) + zi
                plsc.store_scatter(ob, [obase], by1, mask=lane0)
                plsc.store_scatter(ob, [obase + 1], bx1, mask=lane0)
                plsc.store_scatter(ob, [obase + 2], by2, mask=lane0)
                plsc.store_scatter(ob, [obase + 3], bx2, mask=lane0)
                return kept + jnp.where(sup, 0, 1), l2v0n, l2v1n

            kept2, l2v0b, l2v1b = lax.cond(
                valid, do_select, lambda k, a, bb: (k, a, bb), kept, l2v0, l2v1)
            alive2 = jnp.where(valid, alive, 0)
            return kept2, alive2, l2v0b, l2v1b

        kept_fin, _, _, _ = lax.while_loop(
            loop_cond, loop_body,
            (jnp.int32(0), jnp.int32(1), l2[0], l2[1]))

        nvv[...] = jnp.where(iota == 0, kept_fin, 0)
        pltpu.sync_copy(nvv.at[pl.ds(0, 8)], nv_hbm.at[b])
        pltpu.sync_copy(ob.at[pl.ds(0, MAX_OUT * 4)], ob_hbm.at[b])


_nms_call = functools.partial(
    pl.kernel,
    out_type=(jax.ShapeDtypeStruct((NUM_IMAGES, MAX_OUT * 4), jnp.float32),
              jax.ShapeDtypeStruct((NUM_IMAGES, 8), jnp.int32)),
    mesh=plsc.VectorSubcoreMesh(core_axis_name="c", subcore_axis_name="s"),
    compiler_params=pltpu.CompilerParams(
        needs_layout_passes=False, use_tc_tiling_on_sc=False),
    scratch_types=[
        pltpu.VMEM((6 * PAD_N,), jnp.float32),
        pltpu.VMEM((PAD_N,), jnp.float32),
        pltpu.VMEM((PAD_N,), jnp.int32),
        pltpu.VMEM((512,), jnp.float32),
        pltpu.VMEM((N_CHUNKS,), jnp.int32),
        pltpu.VMEM((KPAD,), jnp.float32),
        pltpu.VMEM((KPAD,), jnp.float32),
        pltpu.VMEM((KPAD,), jnp.float32),
        pltpu.VMEM((KPAD,), jnp.float32),
        pltpu.VMEM((OB_PAD,), jnp.float32),
        pltpu.VMEM((16,), jnp.int32),
        pltpu.SemaphoreType.DMA,
    ],
)(_nms_body)


def kernel(rpn_probs, rpn_deltas, anchors):
    dt = jnp.transpose(rpn_deltas, (2, 0, 1))
    sp = rpn_probs[:, :, 1]
    planes = _decode_call(anchors, dt, sp)
    ob, nv = _nms_call(planes)
    selected_boxes = ob.reshape(NUM_IMAGES * MAX_OUT, 4)
    selected_boxes_indices = jnp.repeat(
        jnp.arange(NUM_IMAGES, dtype=jnp.int32), MAX_OUT)
    num_valid = nv[:, 0]
    return selected_boxes, selected_boxes_indices, num_valid
